# Initial kernel scaffold; baseline (speedup 1.0000x reference)
import jax
import jax.numpy as jnp
from jax.experimental import pallas as pl


def kernel(x1, edges, hop, edges2, drug_x, drug_edge_index, drug_batch, params):
    raise NotImplementedError("write your pallas kernel here")


# speedup vs baseline: 3.2122x; 3.2122x over previous
import functools

import jax
import jax.numpy as jnp
import numpy as np
from jax.experimental import pallas as pl
from jax.experimental.pallas import tpu as pltpu

N_DRUGS = 218
N_NODES = 489
BN_EPS = 1e-5
F32 = jnp.float32
BF16 = jnp.bfloat16

ND = 6540
NDP = 6656
ROWB = 512


def _pad2(a, r, c):
    return jnp.zeros((r, c), a.dtype).at[: a.shape[0], : a.shape[1]].set(a)


def _pad1(a, n):
    return jnp.zeros((n,), a.dtype).at[: a.shape[0]].set(a)


def _bdot(a, b):
    return jax.lax.dot(a.astype(BF16), b.astype(BF16),
                       preferred_element_type=F32)


def _drug_l1_body(x_ref, w_ref, dinv_ref, u_ref):
    u_ref[...] = dinv_ref[...][:, None] * _bdot(x_ref[...], w_ref[...])


def _drug_mid_body(agg_ref, u_ref, dinv_ref, b_ref, w_ref, uo_ref, x_ref):
    dinv = dinv_ref[...][:, None]
    x = jax.nn.relu(dinv * (agg_ref[...] + u_ref[...]) + b_ref[...][None, :])
    x_ref[...] = x
    uo_ref[...] = dinv * _bdot(x, w_ref[...])


def _drug_fin_body(agg_ref, u_ref, dinv_ref, b_ref, x_ref):
    dinv = dinv_ref[...][:, None]
    x_ref[...] = jax.nn.relu(dinv * (agg_ref[...] + u_ref[...])
                             + b_ref[...][None, :])


def _row_spec(c):
    return pl.BlockSpec((ROWB, c), lambda i: (i, 0))


def _vec_spec(n):
    return pl.BlockSpec((n,), lambda i: (0,))


def _full_spec(r, c):
    return pl.BlockSpec((r, c), lambda i: (0, 0))


def _drug_l1(x, w, dinv, fin, fout):
    return pl.pallas_call(
        _drug_l1_body,
        grid=(NDP // ROWB,),
        in_specs=[_row_spec(fin), _full_spec(fin, fout), pl.BlockSpec((ROWB,), lambda i: (i,))],
        out_specs=_row_spec(fout),
        out_shape=jax.ShapeDtypeStruct((NDP, fout), F32),
    )(x, w, dinv)


def _drug_mid(agg, u, dinv, b, w, fin, fout):
    return pl.pallas_call(
        _drug_mid_body,
        grid=(NDP // ROWB,),
        in_specs=[_row_spec(fin), _row_spec(fin), pl.BlockSpec((ROWB,), lambda i: (i,)),
                  _vec_spec(fin), _full_spec(fin, fout)],
        out_specs=[_row_spec(fout), _row_spec(fin)],
        out_shape=[jax.ShapeDtypeStruct((NDP, fout), F32),
                   jax.ShapeDtypeStruct((NDP, fin), F32)],
    )(agg, u, dinv, b, w)


def _drug_fin(agg, u, dinv, b, fin):
    return pl.pallas_call(
        _drug_fin_body,
        grid=(NDP // ROWB,),
        in_specs=[_row_spec(fin), _row_spec(fin), pl.BlockSpec((ROWB,), lambda i: (i,)),
                  _vec_spec(fin)],
        out_specs=_row_spec(fin),
        out_shape=jax.ShapeDtypeStruct((NDP, fin), F32),
    )(agg, u, dinv, b)


def _middle_body(pooled_ref, x1_ref, c_ref, wfc_ref, bfc_ref,
                 wg_ref, bg_ref, sel_ref, fc1w_ref, fc1b_ref,
                 w0t_ref, w0b_ref, a_ref, b_ref):
    C = c_ref[...]
    deg = jnp.sum(C, axis=1)
    dinv = jnp.where(deg > 0, jax.lax.rsqrt(deg), 0.0)
    Adj = dinv[:, None] * C * dinv[None, :]

    gfeat = jax.nn.relu(_bdot(pooled_ref[...], wfc_ref[...])
                        + bfc_ref[...][None, :])
    rows = jax.lax.broadcasted_iota(jnp.int32, (512, 1), 0)
    xcat = x1_ref[...] + jnp.where(rows < N_DRUGS, _pad_rows(gfeat, 512), 0.0)

    sel = sel_ref[...][:, None]
    xsel = jnp.zeros((512, 512), F32)
    for l in range(3):
        xl = jax.nn.relu(_bdot(Adj.astype(F32), _bdot(xcat, wg_ref[l]))
                         + bg_ref[l][None, :])
        xsel = xsel + jnp.where(sel == l, xl, 0.0)
    xf = jax.nn.relu(_bdot(xsel, fc1w_ref[...]) + fc1b_ref[...][None, :])
    a_ref[...] = _bdot(xf, w0t_ref[0]) + _bdot(xcat, w0t_ref[1])
    b_ref[...] = _bdot(xf, w0b_ref[0]) + _bdot(xcat, w0b_ref[1])


def _pad_rows(a, n):
    return jnp.pad(a, ((0, n - a.shape[0]), (0, 0)))


def _middle(pooled, x1p, C, wfc, bfc, wg, bg, sel, fc1w, fc1b, w0t, w0b):
    fs = _full_spec
    return pl.pallas_call(
        _middle_body,
        grid=(1,),
        in_specs=[fs(224, 384), fs(512, 512), fs(512, 512), fs(384, 512),
                  _vec_spec(512), pl.BlockSpec((3, 512, 512), lambda i: (0, 0, 0)),
                  pl.BlockSpec((3, 512), lambda i: (0, 0)),
                  pl.BlockSpec((512,), lambda i: (0,)), fs(512, 512),
                  _vec_spec(512), pl.BlockSpec((2, 512, 512), lambda i: (0, 0, 0)),
                  pl.BlockSpec((2, 512, 512), lambda i: (0, 0, 0))],
        out_specs=[fs(512, 512), fs(512, 512)],
        out_shape=[jax.ShapeDtypeStruct((512, 512), F32),
                   jax.ShapeDtypeStruct((512, 512), F32)],
    )(pooled, x1p, C, wfc, bfc, wg, bg, sel, fc1w, fc1b, w0t, w0b)


def _mlp_tail(z0, w1_ref, b1_ref, w2_ref, b2_ref, wl_ref, bl_ref):
    h = jax.nn.relu(z0)
    h = jax.nn.relu(_bdot(h, w1_ref[...]) + b1_ref[...][None, :])
    h = jax.nn.relu(_bdot(h, w2_ref[...]) + b2_ref[...][None, :])
    logit = jnp.sum(h * wl_ref[...][None, :], axis=1) + bl_ref[0]
    return jax.nn.sigmoid(logit)


def _out2_body(a2_ref, b2_ref, b0_ref, w1_ref, b1_ref, w2_ref, b2w_ref,
               wl_ref, bl_ref, o_ref, *, bi):
    z0 = (b2_ref[...][:, None, :] + a2_ref[...][None, :, :]
          + b0_ref[...][None, None, :]).reshape(bi * 272, 512)
    o_ref[...] = _mlp_tail(z0, w1_ref, b1_ref, w2_ref, b2w_ref,
                           wl_ref, bl_ref).reshape(bi, 272)


def _out2(a2, b2, b0, w1, b1, w2, b2w, wl, bl, bi=16):
    nblk = 224 // bi
    return pl.pallas_call(
        functools.partial(_out2_body, bi=bi),
        grid=(nblk,),
        in_specs=[_full_spec(272, 512), pl.BlockSpec((bi, 512), lambda i: (i, 0)),
                  _vec_spec(512), _full_spec(512, 512), _vec_spec(512),
                  _full_spec(512, 512), _vec_spec(512), _vec_spec(512),
                  _vec_spec(8)],
        out_specs=pl.BlockSpec((bi, 272), lambda i: (i, 0)),
        out_shape=jax.ShapeDtypeStruct((224, 272), F32),
    )(a2, b2, b0, w1, b1, w2, b2w, wl, bl)


def _out1_body(z0_ref, w1_ref, b1_ref, w2_ref, b2_ref, wl_ref, bl_ref, o_ref):
    o_ref[...] = _mlp_tail(z0_ref[...], w1_ref, b1_ref, w2_ref, b2_ref,
                           wl_ref, bl_ref)


def _out1(z0, w1, b1, w2, b2, wl, bl):
    return pl.pallas_call(
        _out1_body,
        grid=(8,),
        in_specs=[pl.BlockSpec((1024, 512), lambda i: (i, 0)),
                  _full_spec(512, 512), _vec_spec(512), _full_spec(512, 512),
                  _vec_spec(512), _vec_spec(512), _vec_spec(8)],
        out_specs=pl.BlockSpec((1024,), lambda i: (i,)),
        out_shape=jax.ShapeDtypeStruct((8192,), F32),
    )(z0, w1, b1, w2, b2, wl, bl)


def kernel(x1, edges, hop, edges2, drug_x, drug_edge_index, drug_batch, params):
    p = params
    s, dd = drug_edge_index[0], drug_edge_index[1]

    inv = 1.0 / np.sqrt(1.0 + BN_EPS)
    g0, g1, g2 = p['bn_g0'] * inv, p['bn_g1'] * inv, p['bn_g2'] * inv
    w1p = _pad2(g0[:, None] * p['d_W1'], 512, 512)
    b1p = _pad1(p['bn_b0'] @ p['d_W1'] + p['d_b1'], 512)
    w2p = _pad2(g1[:, None] * p['d_W2'], 512, 512)
    b2p = _pad1(p['bn_b1'] @ p['d_W2'] + p['d_b2'], 512)
    wlp = _pad1((g2[:, None] * p['d_Wl'])[:, 0], 512)
    blp = _pad1(p['bn_b2'] @ p['d_Wl'] + p['d_bl'], 8)
    b0p = _pad1(p['d_b0'], 512)

    gw1 = _pad2(p['g_W1'], 128, 128)
    gw2 = _pad2(p['g_W2'], 128, 256)
    gw3 = _pad2(p['g_W3'], 256, 384)
    gwfc = _pad2(p['g_Wfc'], 384, 512)
    gb1 = _pad1(p['g_b1'], 128)
    gb2 = _pad1(p['g_b2'], 256)
    gb3 = _pad1(p['g_b3'], 384)
    gbfc = _pad1(p['g_bfc'], 512)
    wg = jnp.stack([_pad2(p['W_g%d' % l], 512, 512) for l in range(3)])
    bg = jnp.stack([_pad1(p['b_g%d' % l], 512) for l in range(3)])
    fc1w = _pad2(p['fc1_W'], 512, 512)
    fc1b = _pad1(p['fc1_b'], 512)
    w0t = jnp.stack([_pad2(p['d_W0'][:489], 512, 512),
                     _pad2(p['d_W0'][489:978], 512, 512)])
    w0b = jnp.stack([_pad2(p['d_W0'][978:978 + 489], 512, 512),
                     _pad2(p['d_W0'][978 + 489:], 512, 512)])

    deg = jnp.zeros((ND,), F32).at[dd].add(1.0) + 1.0
    dinv = _pad1(deg ** -0.5, NDP)

    xq = _pad2(drug_x, NDP, 128)
    u1 = _drug_l1(xq, gw1, dinv, 128, 128)
    agg1 = jnp.zeros_like(u1).at[dd].add(u1[s])
    u2, _ = _drug_mid(agg1, u1, dinv, gb1, gw2, 128, 256)
    agg2 = jnp.zeros_like(u2).at[dd].add(u2[s])
    u3, _ = _drug_mid(agg2, u2, dinv, gb2, gw3, 256, 384)
    agg3 = jnp.zeros_like(u3).at[dd].add(u3[s])
    x4 = _drug_fin(agg3, u3, dinv, gb3, 384)

    pooled = jax.ops.segment_max(x4[:ND], drug_batch, num_segments=N_DRUGS)
    pooled = jnp.where(jnp.isfinite(pooled), pooled, 0.0)
    pooled = _pad2(pooled, 224, 384)

    C = (jnp.zeros((512, 512), F32).at[edges[1], edges[0]].add(1.0)
         .at[jnp.arange(N_NODES), jnp.arange(N_NODES)].add(1.0))

    x1p = _pad2(x1, 512, 512)
    sel = _pad1(jnp.where(hop == 0, 2, hop - 1).astype(jnp.int32), 512)
    A, B = _middle(pooled, x1p, C, gwfc, gbfc, wg, bg, sel, fc1w, fc1b,
                   w0t, w0b)

    a2 = _pad_rows(A[N_DRUGS:N_NODES], 272)
    b2 = B[:224]
    out2 = _out2(a2, b2, b0p, w1p, b1p, w2p, b2p, wlp, blp)[:N_DRUGS, :271]

    z0 = A[edges2[1]] + B[edges2[0]] + b0p[None, :]
    out1 = _out1(z0, w1p, b1p, w2p, b2p, wlp, blp)

    return out1, out2
